# Initial kernel scaffold; baseline (speedup 1.0000x reference)
import jax
import jax.numpy as jnp
from jax.experimental import pallas as pl


def kernel(head_enc, tail_enc, ufeat, ifeat, head_id, tail_id, W, W_fc, b_fc, Q, a_comb):
    raise NotImplementedError("write your pallas kernel here")


# speedup vs baseline: 5.8285x; 5.8285x over previous
import jax
import jax.numpy as jnp
from jax import lax
from jax.experimental import pallas as pl
from jax.experimental.pallas import tpu as pltpu
from jax.experimental.pallas import tpu_sc as plsc

N = 10000
E = 320000
RR = 5
EPC = E // RR
D = 128
DO = 64
B = 16384
NBLK = EPC // 128
NTILES = 16
ROWS_A = 640
ROWS_B = 400


def _sc_mesh():
    return plsc.VectorSubcoreMesh(core_axis_name="c", subcore_axis_name="s")


def _split_rows(sid, do_copy):
    @pl.when(sid < 15)
    def _main():
        do_copy(sid * ROWS_A, ROWS_A)

    @pl.when(sid == 15)
    def _tail():
        do_copy(15 * ROWS_A, ROWS_B)


def _degrees_sc(hist_hbm, ones_hbm, zeros_hbm, deg_hbm, shared, iv, ones_v):
    cid = lax.axis_index("c")
    sid = lax.axis_index("s")
    pltpu.sync_copy(ones_hbm, ones_v)

    def zero_my(st, cnt):
        pltpu.sync_copy(zeros_hbm.at[pl.ds(0, cnt)],
                        shared.at[pl.ds(st, cnt)])
    _split_rows(sid, zero_my)
    plsc.subcore_barrier()

    def hist_body(hl, carry):
        h = cid * 10 + hl

        def scat_blk(j):
            off = pl.multiple_of(h * EPC + j * 128, 128)
            pltpu.sync_copy(hist_hbm.at[pl.ds(off, 128)], iv)
            pltpu.sync_copy(ones_v, shared.at[iv], add=True)

        def blk(k, c2):
            scat_blk(sid + 16 * k)
            return c2
        lax.fori_loop(0, 31, blk, 0)

        @pl.when(sid < 4)
        def _tail():
            scat_blk(496 + sid)

        plsc.subcore_barrier()

        def out_my(st, cnt):
            pltpu.sync_copy(shared.at[pl.ds(st, cnt)],
                            deg_hbm.at[h, pl.ds(st, cnt)])
            pltpu.sync_copy(zeros_hbm.at[pl.ds(0, cnt)],
                            shared.at[pl.ds(st, cnt)])
        _split_rows(sid, out_my)
        plsc.subcore_barrier()
        return carry
    lax.fori_loop(0, 10, hist_body, 0)


def _segsum_sc(ftab_hbm, srcb_hbm, dstb_hbm, zeros_hbm, s_hbm,
               shared, siv, div, rows, sem):
    cid = lax.axis_index("c")
    sid = lax.axis_index("s")

    def zero_my(start, cnt):
        pltpu.sync_copy(zeros_hbm.at[pl.ds(0, cnt)],
                        shared.at[pl.ds(start, cnt)])
    _split_rows(sid, zero_my)
    plsc.subcore_barrier()

    def chunk_body(hl, carry):
        q = cid * 5 + hl

        def edge_blk(j):
            off = pl.multiple_of(q * EPC + j * 128, 128)
            pltpu.sync_copy(srcb_hbm.at[pl.ds(off, 128)], siv)
            pltpu.sync_copy(dstb_hbm.at[pl.ds(off, 128)], div)
            pltpu.async_copy(ftab_hbm.at[siv], rows, sem).wait()
            pltpu.sync_copy(rows, shared.at[div], add=True)

        def blk(k, c2):
            edge_blk(sid + 16 * k)
            return c2
        lax.fori_loop(0, 31, blk, 0)

        @pl.when(sid < 4)
        def _tail():
            edge_blk(496 + sid)

        plsc.subcore_barrier()

        def out_my(start, cnt):
            pltpu.sync_copy(shared.at[pl.ds(start, cnt)],
                            s_hbm.at[q, pl.ds(start, cnt)])
            pltpu.sync_copy(zeros_hbm.at[pl.ds(0, cnt)],
                            shared.at[pl.ds(start, cnt)])
        _split_rows(sid, out_my)
        plsc.subcore_barrier()
        return carry
    lax.fori_loop(0, 5, chunk_body, 0)


def _pairgather_sc(uv_hbm, idx_hbm, out_hbm, iv, rows, sem):
    cid = lax.axis_index("c")
    sid = lax.axis_index("s")
    wid = cid * NTILES + sid

    def blk(k, carry):
        j = wid * 8 + k
        off = pl.multiple_of(j * 128, 128)
        pltpu.sync_copy(idx_hbm.at[pl.ds(off, 128)], iv)
        pltpu.async_copy(uv_hbm.at[iv], rows, sem).wait()
        pltpu.sync_copy(rows, out_hbm.at[pl.ds(off, 128)])
        return carry
    lax.fori_loop(0, 8, blk, 0)


def _prescale_tc(deg_ref, feats_ref, o_ref):
    d = deg_ref[0][:, :1]
    ci = lax.rsqrt(jnp.maximum(d, 1.0))
    o_ref[0] = feats_ref[0] * ci


def _combine_tc(s_ref, deg_ref, w_ref, wfc_ref, bfc_ref, o_ref):
    acc = jnp.zeros((2048, D), jnp.float32)
    for r in range(RR):
        d = deg_ref[0, r][:, :1]
        cj = lax.rsqrt(jnp.maximum(d, 1.0))
        acc = acc + jnp.dot(s_ref[0, r] * cj, w_ref[r],
                            preferred_element_type=jnp.float32)
    h = jnp.where(acc > 0, acc, 0.1 * acc)
    z = jnp.dot(h, wfc_ref[...], preferred_element_type=jnp.float32) + bfc_ref[0]
    z = jnp.where(z > 0, z, 0.1 * z)
    o_ref[0] = jnp.concatenate([z, jnp.zeros((2048, D - DO), jnp.float32)], axis=1)


def _decoder_tc(u_ref, v_ref, q_ref, a_ref, o_ref):
    u = u_ref[:, :DO]
    v = v_ref[:, :DO]
    b0 = jnp.sum(jnp.dot(u, q_ref[0], preferred_element_type=jnp.float32) * v,
                 axis=1)
    b1 = jnp.sum(jnp.dot(u, q_ref[1], preferred_element_type=jnp.float32) * v,
                 axis=1)
    o_ref[...] = b0[:, None] * a_ref[0][None, :] + b1[:, None] * a_ref[1][None, :]


def kernel(head_enc, tail_enc, ufeat, ifeat, head_id, tail_id, W, W_fc, b_fc,
           Q, a_comb):
    f32 = jnp.float32
    src_all = jnp.concatenate([head_enc[0], tail_enc[0]]).astype(jnp.int32)
    dst_all = jnp.concatenate([head_enc[1], tail_enc[1]]).astype(jnp.int32)
    qoff = jnp.repeat(jnp.arange(10, dtype=jnp.int32) * N, EPC)
    srcb = src_all + qoff
    dstb = dst_all
    hist_idx = jnp.concatenate([src_all, dst_all])
    feats = jnp.stack([ifeat, ufeat])
    ones16 = jnp.ones((128, 16), f32)
    zeros16 = jnp.zeros((ROWS_A, 16), f32)
    zeros128 = jnp.zeros((ROWS_A, D), f32)
    ones128 = jnp.ones((128, D), f32)
    pair_idx = jnp.concatenate([head_id.astype(jnp.int32),
                                tail_id.astype(jnp.int32) + N])

    mesh = _sc_mesh()

    deg2 = pl.kernel(
        _degrees_sc,
        out_type=jax.ShapeDtypeStruct((20, N, D), f32),
        mesh=mesh,
        scratch_types=[
            pltpu.VMEM_SHARED((N, D), f32),
            pltpu.VMEM((128,), jnp.int32),
            pltpu.VMEM((128, D), f32),
        ],
    )(hist_idx, ones128, zeros128)

    degsrc = deg2[:10]
    degdst = deg2[10:].reshape(2, RR, N, D)

    fscaled = pl.pallas_call(
        _prescale_tc,
        grid=(10, 5),
        in_specs=[
            pl.BlockSpec((1, 2048, D), lambda q, b: (q, b, 0)),
            pl.BlockSpec((1, 2048, D), lambda q, b: (q // 5, b, 0)),
        ],
        out_specs=pl.BlockSpec((1, 2048, D), lambda q, b: (q, b, 0)),
        out_shape=jax.ShapeDtypeStruct((10, N, D), f32),
    )(degsrc, feats)

    S = pl.kernel(
        _segsum_sc,
        out_type=jax.ShapeDtypeStruct((10, N, D), f32),
        mesh=mesh,
        scratch_types=[
            pltpu.VMEM_SHARED((N, D), f32),
            pltpu.VMEM((128,), jnp.int32),
            pltpu.VMEM((128,), jnp.int32),
            pltpu.VMEM((128, D), f32),
            pltpu.SemaphoreType.DMA,
        ],
    )(fscaled.reshape(10 * N, D), srcb, dstb, zeros128)

    outs = pl.pallas_call(
        _combine_tc,
        grid=(2, 5),
        in_specs=[
            pl.BlockSpec((1, RR, 2048, D), lambda c, b: (c, 0, b, 0)),
            pl.BlockSpec((1, RR, 2048, D), lambda c, b: (c, 0, b, 0)),
            pl.BlockSpec((RR, D, D), lambda c, b: (0, 0, 0)),
            pl.BlockSpec((D, DO), lambda c, b: (0, 0)),
            pl.BlockSpec((1, DO), lambda c, b: (0, 0)),
        ],
        out_specs=pl.BlockSpec((1, 2048, D), lambda c, b: (c, b, 0)),
        out_shape=jax.ShapeDtypeStruct((2, N, D), f32),
    )(S.reshape(2, RR, N, D), degdst, W, W_fc, b_fc.reshape(1, DO))

    uv = pl.kernel(
        _pairgather_sc,
        out_type=jax.ShapeDtypeStruct((2 * B, D), f32),
        mesh=mesh,
        scratch_types=[
            pltpu.VMEM((128,), jnp.int32),
            pltpu.VMEM((128, D), f32),
            pltpu.SemaphoreType.DMA,
        ],
    )(outs.reshape(2 * N, D), pair_idx)

    pred = pl.pallas_call(
        _decoder_tc,
        grid=(8,),
        in_specs=[
            pl.BlockSpec((2048, D), lambda i: (i, 0)),
            pl.BlockSpec((2048, D), lambda i: (i, 0)),
            pl.BlockSpec((2, DO, DO), lambda i: (0, 0, 0)),
            pl.BlockSpec((2, RR), lambda i: (0, 0)),
        ],
        out_specs=pl.BlockSpec((2048, RR), lambda i: (i, 0)),
        out_shape=jax.ShapeDtypeStruct((B, RR), f32),
    )(uv[:B], uv[B:], Q, a_comb)

    return pred


# speedup vs baseline: 6.9450x; 1.1916x over previous
import jax
import jax.numpy as jnp
from jax import lax
from jax.experimental import pallas as pl
from jax.experimental.pallas import tpu as pltpu
from jax.experimental.pallas import tpu_sc as plsc

N = 10000
E = 320000
RR = 5
EPC = E // RR
D = 128
DO = 64
B = 16384
NBLK = EPC // 128
NTILES = 16
ROWS_A = 640
ROWS_B = 400


def _sc_mesh():
    return plsc.VectorSubcoreMesh(core_axis_name="c", subcore_axis_name="s")


def _split_rows(sid, do_copy):
    @pl.when(sid < 15)
    def _main():
        do_copy(sid * ROWS_A, ROWS_A)

    @pl.when(sid == 15)
    def _tail():
        do_copy(15 * ROWS_A, ROWS_B)


def _degrees_sc(hist_hbm, ones_hbm, zeros_hbm, deg_hbm, shared, iv, ones_v):
    cid = lax.axis_index("c")
    sid = lax.axis_index("s")
    pltpu.sync_copy(ones_hbm, ones_v)

    def zero_my(st, cnt):
        pltpu.sync_copy(zeros_hbm.at[pl.ds(0, cnt)],
                        shared.at[pl.ds(st, cnt)])
    _split_rows(sid, zero_my)
    plsc.subcore_barrier()

    def hist_body(hl, carry):
        h = cid * 10 + hl

        def scat_blk(j):
            off = pl.multiple_of(h * EPC + j * 128, 128)
            pltpu.sync_copy(hist_hbm.at[pl.ds(off, 128)], iv)
            pltpu.sync_copy(ones_v, shared.at[iv], add=True)

        def blk(k, c2):
            scat_blk(sid + 16 * k)
            return c2
        lax.fori_loop(0, 31, blk, 0)

        @pl.when(sid < 4)
        def _tail():
            scat_blk(496 + sid)

        plsc.subcore_barrier()

        def out_my(st, cnt):
            pltpu.sync_copy(shared.at[pl.ds(st, cnt)],
                            deg_hbm.at[h, pl.ds(st, cnt)])
            pltpu.sync_copy(zeros_hbm.at[pl.ds(0, cnt)],
                            shared.at[pl.ds(st, cnt)])
        _split_rows(sid, out_my)
        plsc.subcore_barrier()
        return carry
    lax.fori_loop(0, 10, hist_body, 0)


def _segsum_sc(ftab_hbm, srcb_hbm, dstb_hbm, zeros_hbm, s_hbm,
               shared, siv, div, rows, sems):
    cid = lax.axis_index("c")
    sid = lax.axis_index("s")

    def zero_my(start, cnt):
        pltpu.sync_copy(zeros_hbm.at[pl.ds(0, cnt)],
                        shared.at[pl.ds(start, cnt)])
    _split_rows(sid, zero_my)
    plsc.subcore_barrier()

    NBK = 50
    BW = 80

    def chunk_body(hl, carry):
        q = cid * 5 + hl
        base = pl.multiple_of(q * EPC + sid * (EPC // 16), 8)

        def load_gather_start(k, buf):
            off = pl.multiple_of(base + k * BW, 8)
            pltpu.sync_copy(srcb_hbm.at[pl.ds(off, BW)], siv.at[buf])
            pltpu.async_copy(ftab_hbm.at[siv.at[buf]], rows.at[buf],
                             sems.at[buf])

        load_gather_start(0, 0)

        def blk(k, c2):
            buf = lax.rem(k, 2)

            @pl.when(k < NBK - 1)
            def _prefetch():
                load_gather_start(k + 1, lax.rem(k + 1, 2))

            off = pl.multiple_of(base + k * BW, 8)
            pltpu.sync_copy(dstb_hbm.at[pl.ds(off, BW)], div)
            pltpu.make_async_copy(ftab_hbm.at[siv.at[buf]], rows.at[buf],
                                  sems.at[buf]).wait()
            pltpu.sync_copy(rows.at[buf], shared.at[div], add=True)
            return c2
        lax.fori_loop(0, NBK, blk, 0)

        plsc.subcore_barrier()

        def out_my(start, cnt):
            pltpu.sync_copy(shared.at[pl.ds(start, cnt)],
                            s_hbm.at[q, pl.ds(start, cnt)])
            pltpu.sync_copy(zeros_hbm.at[pl.ds(0, cnt)],
                            shared.at[pl.ds(start, cnt)])
        _split_rows(sid, out_my)
        plsc.subcore_barrier()
        return carry
    lax.fori_loop(0, 5, chunk_body, 0)


def _pairgather_sc(uv_hbm, idx_hbm, out_hbm, iv, rows, sem):
    cid = lax.axis_index("c")
    sid = lax.axis_index("s")
    wid = cid * NTILES + sid

    def blk(k, carry):
        j = wid * 8 + k
        off = pl.multiple_of(j * 128, 128)
        pltpu.sync_copy(idx_hbm.at[pl.ds(off, 128)], iv)
        pltpu.async_copy(uv_hbm.at[iv], rows, sem).wait()
        pltpu.sync_copy(rows, out_hbm.at[pl.ds(off, 128)])
        return carry
    lax.fori_loop(0, 8, blk, 0)


def _prescale_tc(deg_ref, feats_ref, o_ref):
    d = deg_ref[0][:, :1]
    ci = lax.rsqrt(jnp.maximum(d, 1.0))
    o_ref[0] = feats_ref[0] * ci


def _combine_tc(s_ref, deg_ref, w_ref, wfc_ref, bfc_ref, o_ref):
    acc = jnp.zeros((2048, D), jnp.float32)
    for r in range(RR):
        d = deg_ref[r][:, :1]
        cj = lax.rsqrt(jnp.maximum(d, 1.0))
        acc = acc + jnp.dot(s_ref[r] * cj, w_ref[r],
                            preferred_element_type=jnp.float32)
    h = jnp.where(acc > 0, acc, 0.1 * acc)
    z = jnp.dot(h, wfc_ref[...], preferred_element_type=jnp.float32) + bfc_ref[0]
    z = jnp.where(z > 0, z, 0.1 * z)
    o_ref[0] = jnp.concatenate([z, jnp.zeros((2048, D - DO), jnp.float32)], axis=1)


def _decoder_tc(u_ref, v_ref, q_ref, a_ref, o_ref):
    u = u_ref[:, :DO]
    v = v_ref[:, :DO]
    b0 = jnp.sum(jnp.dot(u, q_ref[0], preferred_element_type=jnp.float32) * v,
                 axis=1)
    b1 = jnp.sum(jnp.dot(u, q_ref[1], preferred_element_type=jnp.float32) * v,
                 axis=1)
    o_ref[...] = b0[:, None] * a_ref[0][None, :] + b1[:, None] * a_ref[1][None, :]


def kernel(head_enc, tail_enc, ufeat, ifeat, head_id, tail_id, W, W_fc, b_fc,
           Q, a_comb):
    f32 = jnp.float32
    src_all = jnp.concatenate([head_enc[0], tail_enc[0]]).astype(jnp.int32)
    dst_all = jnp.concatenate([head_enc[1], tail_enc[1]]).astype(jnp.int32)
    qoff = jnp.repeat(jnp.arange(10, dtype=jnp.int32) * N, EPC)
    srcb = src_all + qoff
    dstb = dst_all
    hist_idx = jnp.concatenate([src_all, dst_all])
    feats = jnp.stack([ifeat, ufeat])
    ones16 = jnp.ones((128, 16), f32)
    zeros16 = jnp.zeros((ROWS_A, 16), f32)
    zeros128 = jnp.zeros((ROWS_A, D), f32)
    ones128 = jnp.ones((128, D), f32)
    pair_idx = jnp.concatenate([head_id.astype(jnp.int32),
                                tail_id.astype(jnp.int32) + N])

    mesh = _sc_mesh()

    deg2 = pl.kernel(
        _degrees_sc,
        out_type=jax.ShapeDtypeStruct((20, N, D), f32),
        mesh=mesh,
        scratch_types=[
            pltpu.VMEM_SHARED((N, D), f32),
            pltpu.VMEM((128,), jnp.int32),
            pltpu.VMEM((128, D), f32),
        ],
    )(hist_idx, ones128, zeros128)


    fscaled = pl.pallas_call(
        _prescale_tc,
        grid=(10, 5),
        in_specs=[
            pl.BlockSpec((1, 2048, D), lambda q, b: (q, b, 0)),
            pl.BlockSpec((1, 2048, D), lambda q, b: (q // 5, b, 0)),
        ],
        out_specs=pl.BlockSpec((1, 2048, D), lambda q, b: (q, b, 0)),
        out_shape=jax.ShapeDtypeStruct((10, N, D), f32),
    )(deg2, feats)

    S = pl.kernel(
        _segsum_sc,
        out_type=jax.ShapeDtypeStruct((10, N, D), f32),
        mesh=mesh,
        scratch_types=[
            pltpu.VMEM_SHARED((N, D), f32),
            pltpu.VMEM((2, 80), jnp.int32),
            pltpu.VMEM((80,), jnp.int32),
            pltpu.VMEM((2, 80, D), f32),
            pltpu.SemaphoreType.DMA((2,)),
        ],
    )(fscaled.reshape(10 * N, D), srcb, dstb, zeros128)

    outs = pl.pallas_call(
        _combine_tc,
        grid=(2, 5),
        in_specs=[
            pl.BlockSpec((RR, 2048, D), lambda c, b: (c, b, 0)),
            pl.BlockSpec((RR, 2048, D), lambda c, b: (2 + c, b, 0)),
            pl.BlockSpec((RR, D, D), lambda c, b: (0, 0, 0)),
            pl.BlockSpec((D, DO), lambda c, b: (0, 0)),
            pl.BlockSpec((1, DO), lambda c, b: (0, 0)),
        ],
        out_specs=pl.BlockSpec((1, 2048, D), lambda c, b: (c, b, 0)),
        out_shape=jax.ShapeDtypeStruct((2, N, D), f32),
    )(S, deg2, W, W_fc, b_fc.reshape(1, DO))

    uv = pl.kernel(
        _pairgather_sc,
        out_type=jax.ShapeDtypeStruct((2 * B, D), f32),
        mesh=mesh,
        scratch_types=[
            pltpu.VMEM((128,), jnp.int32),
            pltpu.VMEM((128, D), f32),
            pltpu.SemaphoreType.DMA,
        ],
    )(outs.reshape(2 * N, D), pair_idx)

    pred = pl.pallas_call(
        _decoder_tc,
        grid=(8,),
        in_specs=[
            pl.BlockSpec((2048, D), lambda i: (i, 0)),
            pl.BlockSpec((2048, D), lambda i: (i + 8, 0)),
            pl.BlockSpec((2, DO, DO), lambda i: (0, 0, 0)),
            pl.BlockSpec((2, RR), lambda i: (0, 0)),
        ],
        out_specs=pl.BlockSpec((2048, RR), lambda i: (i, 0)),
        out_shape=jax.ShapeDtypeStruct((B, RR), f32),
    )(uv, uv, Q, a_comb)

    return pred


# speedup vs baseline: 7.8948x; 1.1368x over previous
import jax
import jax.numpy as jnp
from jax import lax
from jax.experimental import pallas as pl
from jax.experimental.pallas import tpu as pltpu
from jax.experimental.pallas import tpu_sc as plsc

N = 10000
E = 320000
RR = 5
EPC = E // RR
D = 128
DO = 64
B = 16384
NBLK = EPC // 128
NTILES = 16
ROWS_A = 640
ROWS_B = 400


def _sc_mesh():
    return plsc.VectorSubcoreMesh(core_axis_name="c", subcore_axis_name="s")


def _split_rows(sid, do_copy):
    @pl.when(sid < 15)
    def _main():
        do_copy(sid * ROWS_A, ROWS_A)

    @pl.when(sid == 15)
    def _tail():
        do_copy(15 * ROWS_A, ROWS_B)


def _degrees_sc(hist_hbm, ones_hbm, zeros_hbm, deg_hbm, shared, iv, ones_v, sems):
    cid = lax.axis_index("c")
    sid = lax.axis_index("s")
    pltpu.sync_copy(ones_hbm, ones_v)

    def zero_my(st, cnt):
        pltpu.sync_copy(zeros_hbm.at[pl.ds(0, cnt)],
                        shared.at[pl.ds(st, cnt)])
    _split_rows(sid, zero_my)
    plsc.subcore_barrier()

    NBK = 50
    BW = 80

    def wait_scat(buf):
        pltpu.make_async_copy(ones_v, shared.at[iv.at[buf]],
                              sems.at[buf]).wait()

    def hist_body(hl, carry):
        h = cid * 10 + hl
        base = pl.multiple_of(h * EPC + sid * (EPC // 16), 8)
        pltpu.sync_copy(hist_hbm.at[pl.ds(base, BW)], iv.at[0])

        def blk(k, c2):
            buf = lax.rem(k, 2)
            nbuf = lax.rem(k + 1, 2)
            pltpu.async_copy(ones_v, shared.at[iv.at[buf]], sems.at[buf],
                             add=True)

            @pl.when(k < NBK - 1)
            def _prefetch():
                @pl.when(k >= 1)
                def _drain():
                    wait_scat(nbuf)
                off = pl.multiple_of(base + (k + 1) * BW, 8)
                pltpu.sync_copy(hist_hbm.at[pl.ds(off, BW)], iv.at[nbuf])
            return c2
        lax.fori_loop(0, NBK, blk, 0)
        wait_scat(0)
        wait_scat(1)

        plsc.subcore_barrier()

        def out_my(st, cnt):
            pltpu.sync_copy(shared.at[pl.ds(st, cnt)],
                            deg_hbm.at[h, pl.ds(st, cnt)])
            pltpu.sync_copy(zeros_hbm.at[pl.ds(0, cnt)],
                            shared.at[pl.ds(st, cnt)])
        _split_rows(sid, out_my)
        plsc.subcore_barrier()
        return carry
    lax.fori_loop(0, 10, hist_body, 0)


def _segsum_sc(ftab_hbm, srcb_hbm, dstb_hbm, zeros_hbm, s_hbm,
               shared, siv, div, rows, sems):
    cid = lax.axis_index("c")
    sid = lax.axis_index("s")

    def zero_my(start, cnt):
        pltpu.sync_copy(zeros_hbm.at[pl.ds(0, cnt)],
                        shared.at[pl.ds(start, cnt)])
    _split_rows(sid, zero_my)
    plsc.subcore_barrier()

    NBK = 50
    BW = 80

    def chunk_body(hl, carry):
        q = cid * 5 + hl
        base = pl.multiple_of(q * EPC + sid * (EPC // 16), 8)

        def load_gather_start(k, buf):
            off = pl.multiple_of(base + k * BW, 8)
            pltpu.sync_copy(srcb_hbm.at[pl.ds(off, BW)], siv.at[buf])
            pltpu.async_copy(ftab_hbm.at[siv.at[buf]], rows.at[buf],
                             sems.at[buf])

        load_gather_start(0, 0)

        def blk(k, c2):
            buf = lax.rem(k, 2)

            @pl.when(k < NBK - 1)
            def _prefetch():
                load_gather_start(k + 1, lax.rem(k + 1, 2))

            off = pl.multiple_of(base + k * BW, 8)
            pltpu.sync_copy(dstb_hbm.at[pl.ds(off, BW)], div)
            pltpu.make_async_copy(ftab_hbm.at[siv.at[buf]], rows.at[buf],
                                  sems.at[buf]).wait()
            pltpu.sync_copy(rows.at[buf], shared.at[div], add=True)
            return c2
        lax.fori_loop(0, NBK, blk, 0)

        plsc.subcore_barrier()

        def out_my(start, cnt):
            pltpu.sync_copy(shared.at[pl.ds(start, cnt)],
                            s_hbm.at[q, pl.ds(start, cnt)])
            pltpu.sync_copy(zeros_hbm.at[pl.ds(0, cnt)],
                            shared.at[pl.ds(start, cnt)])
        _split_rows(sid, out_my)
        plsc.subcore_barrier()
        return carry
    lax.fori_loop(0, 5, chunk_body, 0)


def _pairgather_sc(uv_hbm, idx_hbm, out_hbm, iv, rows, sem):
    cid = lax.axis_index("c")
    sid = lax.axis_index("s")
    wid = cid * NTILES + sid

    def blk(k, carry):
        j = wid * 8 + k
        off = pl.multiple_of(j * 128, 128)
        pltpu.sync_copy(idx_hbm.at[pl.ds(off, 128)], iv)
        pltpu.async_copy(uv_hbm.at[iv], rows, sem).wait()
        pltpu.sync_copy(rows, out_hbm.at[pl.ds(off, 128)])
        return carry
    lax.fori_loop(0, 8, blk, 0)


def _prescale_tc(deg_ref, feats_ref, o_ref):
    d = deg_ref[0][:, :1]
    ci = lax.rsqrt(jnp.maximum(d, 1.0))
    o_ref[0] = feats_ref[0] * ci


def _combine_tc(s_ref, deg_ref, w_ref, wfc_ref, bfc_ref, o_ref):
    acc = jnp.zeros((2048, D), jnp.float32)
    for r in range(RR):
        d = deg_ref[r][:, :1]
        cj = lax.rsqrt(jnp.maximum(d, 1.0))
        acc = acc + jnp.dot(s_ref[r] * cj, w_ref[r],
                            preferred_element_type=jnp.float32)
    h = jnp.where(acc > 0, acc, 0.1 * acc)
    z = jnp.dot(h, wfc_ref[...], preferred_element_type=jnp.float32) + bfc_ref[0]
    z = jnp.where(z > 0, z, 0.1 * z)
    o_ref[0] = jnp.concatenate([z, jnp.zeros((2048, D - DO), jnp.float32)], axis=1)


def _decoder_tc(u_ref, v_ref, q_ref, a_ref, o_ref):
    u = u_ref[:, :DO]
    v = v_ref[:, :DO]
    b0 = jnp.sum(jnp.dot(u, q_ref[0], preferred_element_type=jnp.float32) * v,
                 axis=1)
    b1 = jnp.sum(jnp.dot(u, q_ref[1], preferred_element_type=jnp.float32) * v,
                 axis=1)
    o_ref[...] = b0[:, None] * a_ref[0][None, :] + b1[:, None] * a_ref[1][None, :]


def kernel(head_enc, tail_enc, ufeat, ifeat, head_id, tail_id, W, W_fc, b_fc,
           Q, a_comb):
    f32 = jnp.float32
    src_all = jnp.concatenate([head_enc[0], tail_enc[0]]).astype(jnp.int32)
    dst_all = jnp.concatenate([head_enc[1], tail_enc[1]]).astype(jnp.int32)
    qoff = jnp.repeat(jnp.arange(10, dtype=jnp.int32) * N, EPC)
    srcb = src_all + qoff
    dstb = dst_all
    hist_idx = jnp.concatenate([src_all, dst_all])
    feats = jnp.stack([ifeat, ufeat])
    ones16 = jnp.ones((128, 16), f32)
    zeros16 = jnp.zeros((ROWS_A, 16), f32)
    zeros128 = jnp.zeros((ROWS_A, D), f32)
    ones80 = jnp.ones((80, D), f32)
    pair_idx = jnp.concatenate([head_id.astype(jnp.int32),
                                tail_id.astype(jnp.int32) + N])

    mesh = _sc_mesh()

    deg2 = pl.kernel(
        _degrees_sc,
        out_type=jax.ShapeDtypeStruct((20, N, D), f32),
        mesh=mesh,
        scratch_types=[
            pltpu.VMEM_SHARED((N, D), f32),
            pltpu.VMEM((2, 80), jnp.int32),
            pltpu.VMEM((80, D), f32),
            pltpu.SemaphoreType.DMA((2,)),
        ],
    )(hist_idx, ones80, zeros128)


    fscaled = pl.pallas_call(
        _prescale_tc,
        grid=(10, 5),
        in_specs=[
            pl.BlockSpec((1, 2048, D), lambda q, b: (q, b, 0)),
            pl.BlockSpec((1, 2048, D), lambda q, b: (q // 5, b, 0)),
        ],
        out_specs=pl.BlockSpec((1, 2048, D), lambda q, b: (q, b, 0)),
        out_shape=jax.ShapeDtypeStruct((10, N, D), f32),
    )(deg2, feats)

    S = pl.kernel(
        _segsum_sc,
        out_type=jax.ShapeDtypeStruct((10, N, D), f32),
        mesh=mesh,
        scratch_types=[
            pltpu.VMEM_SHARED((N, D), f32),
            pltpu.VMEM((2, 80), jnp.int32),
            pltpu.VMEM((80,), jnp.int32),
            pltpu.VMEM((2, 80, D), f32),
            pltpu.SemaphoreType.DMA((2,)),
        ],
    )(fscaled.reshape(10 * N, D), srcb, dstb, zeros128)

    outs = pl.pallas_call(
        _combine_tc,
        grid=(2, 5),
        in_specs=[
            pl.BlockSpec((RR, 2048, D), lambda c, b: (c, b, 0)),
            pl.BlockSpec((RR, 2048, D), lambda c, b: (2 + c, b, 0)),
            pl.BlockSpec((RR, D, D), lambda c, b: (0, 0, 0)),
            pl.BlockSpec((D, DO), lambda c, b: (0, 0)),
            pl.BlockSpec((1, DO), lambda c, b: (0, 0)),
        ],
        out_specs=pl.BlockSpec((1, 2048, D), lambda c, b: (c, b, 0)),
        out_shape=jax.ShapeDtypeStruct((2, N, D), f32),
    )(S, deg2, W, W_fc, b_fc.reshape(1, DO))

    uv = pl.kernel(
        _pairgather_sc,
        out_type=jax.ShapeDtypeStruct((2 * B, D), f32),
        mesh=mesh,
        scratch_types=[
            pltpu.VMEM((128,), jnp.int32),
            pltpu.VMEM((128, D), f32),
            pltpu.SemaphoreType.DMA,
        ],
    )(outs.reshape(2 * N, D), pair_idx)

    pred = pl.pallas_call(
        _decoder_tc,
        grid=(8,),
        in_specs=[
            pl.BlockSpec((2048, D), lambda i: (i, 0)),
            pl.BlockSpec((2048, D), lambda i: (i + 8, 0)),
            pl.BlockSpec((2, DO, DO), lambda i: (0, 0, 0)),
            pl.BlockSpec((2, RR), lambda i: (0, 0)),
        ],
        out_specs=pl.BlockSpec((2048, RR), lambda i: (i, 0)),
        out_shape=jax.ShapeDtypeStruct((B, RR), f32),
    )(uv, uv, Q, a_comb)

    return pred


# speedup vs baseline: 7.9581x; 1.0080x over previous
import jax
import jax.numpy as jnp
from jax import lax
from jax.experimental import pallas as pl
from jax.experimental.pallas import tpu as pltpu
from jax.experimental.pallas import tpu_sc as plsc

N = 10000
E = 320000
RR = 5
EPC = E // RR
D = 128
DO = 64
B = 16384
NBLK = EPC // 128
NTILES = 16
ROWS_A = 640
ROWS_B = 400


def _sc_mesh():
    return plsc.VectorSubcoreMesh(core_axis_name="c", subcore_axis_name="s")


def _split_rows(sid, do_copy):
    @pl.when(sid < 15)
    def _main():
        do_copy(sid * ROWS_A, ROWS_A)

    @pl.when(sid == 15)
    def _tail():
        do_copy(15 * ROWS_A, ROWS_B)


def _degrees_sc(hist_hbm, ones_hbm, zeros_hbm, deg_hbm, shared, iv, ones_v, sems):
    cid = lax.axis_index("c")
    sid = lax.axis_index("s")
    pltpu.sync_copy(ones_hbm, ones_v)

    def zero_my(st, cnt):
        pltpu.sync_copy(zeros_hbm.at[pl.ds(0, cnt)],
                        shared.at[pl.ds(st, cnt)])
    _split_rows(sid, zero_my)
    plsc.subcore_barrier()

    NBK = 50
    BW = 80

    def wait_scat(buf):
        pltpu.make_async_copy(ones_v, shared.at[iv.at[buf]],
                              sems.at[buf]).wait()

    def hist_body(hl, carry):
        h = cid * 10 + hl
        base = pl.multiple_of(h * EPC + sid * (EPC // 16), 8)
        pltpu.sync_copy(hist_hbm.at[pl.ds(base, BW)], iv.at[0])

        def blk(k, c2):
            buf = lax.rem(k, 2)
            nbuf = lax.rem(k + 1, 2)
            pltpu.async_copy(ones_v, shared.at[iv.at[buf]], sems.at[buf],
                             add=True)

            @pl.when(k < NBK - 1)
            def _prefetch():
                @pl.when(k >= 1)
                def _drain():
                    wait_scat(nbuf)
                off = pl.multiple_of(base + (k + 1) * BW, 8)
                pltpu.sync_copy(hist_hbm.at[pl.ds(off, BW)], iv.at[nbuf])
            return c2
        lax.fori_loop(0, NBK, blk, 0)
        wait_scat(0)
        wait_scat(1)

        plsc.subcore_barrier()

        def out_my(st, cnt):
            pltpu.sync_copy(shared.at[pl.ds(st, cnt)],
                            deg_hbm.at[h, pl.ds(st, cnt)])
            pltpu.sync_copy(zeros_hbm.at[pl.ds(0, cnt)],
                            shared.at[pl.ds(st, cnt)])
        _split_rows(sid, out_my)
        plsc.subcore_barrier()
        return carry
    lax.fori_loop(0, 10, hist_body, 0)


def _segsum_sc(ftab_hbm, srcb_hbm, dstb_hbm, zeros_hbm, s_hbm,
               shared, siv, div, rows, gsem, ssem):
    cid = lax.axis_index("c")
    sid = lax.axis_index("s")

    def zero_my(start, cnt):
        pltpu.sync_copy(zeros_hbm.at[pl.ds(0, cnt)],
                        shared.at[pl.ds(start, cnt)])
    _split_rows(sid, zero_my)
    plsc.subcore_barrier()

    NBK = 50
    BW = 80

    def chunk_body(hl, carry):
        q = cid * 5 + hl
        base = pl.multiple_of(q * EPC + sid * (EPC // 16), 8)

        def load_gather_start(k, buf):
            off = pl.multiple_of(base + k * BW, 8)
            pltpu.sync_copy(srcb_hbm.at[pl.ds(off, BW)], siv.at[buf])
            pltpu.async_copy(ftab_hbm.at[siv.at[buf]], rows.at[buf],
                             gsem.at[buf])

        def wait_scat(buf):
            pltpu.make_async_copy(rows.at[buf], shared.at[div.at[buf]],
                                  ssem.at[buf]).wait()

        load_gather_start(0, 0)

        def blk(k, c2):
            buf = lax.rem(k, 2)
            nbuf = lax.rem(k + 1, 2)

            @pl.when(k < NBK - 1)
            def _prefetch():
                @pl.when(k >= 1)
                def _drain():
                    wait_scat(nbuf)
                load_gather_start(k + 1, nbuf)

            off = pl.multiple_of(base + k * BW, 8)
            pltpu.sync_copy(dstb_hbm.at[pl.ds(off, BW)], div.at[buf])
            pltpu.make_async_copy(ftab_hbm.at[siv.at[buf]], rows.at[buf],
                                  gsem.at[buf]).wait()
            pltpu.async_copy(rows.at[buf], shared.at[div.at[buf]],
                             ssem.at[buf], add=True)
            return c2
        lax.fori_loop(0, NBK, blk, 0)
        wait_scat(0)
        wait_scat(1)

        plsc.subcore_barrier()

        def out_my(start, cnt):
            pltpu.sync_copy(shared.at[pl.ds(start, cnt)],
                            s_hbm.at[q, pl.ds(start, cnt)])
            pltpu.sync_copy(zeros_hbm.at[pl.ds(0, cnt)],
                            shared.at[pl.ds(start, cnt)])
        _split_rows(sid, out_my)
        plsc.subcore_barrier()
        return carry
    lax.fori_loop(0, 5, chunk_body, 0)


def _pairgather_sc(uv_hbm, idx_hbm, out_hbm, iv, rows, sem):
    cid = lax.axis_index("c")
    sid = lax.axis_index("s")
    wid = cid * NTILES + sid

    def blk(k, carry):
        j = wid * 8 + k
        off = pl.multiple_of(j * 128, 128)
        pltpu.sync_copy(idx_hbm.at[pl.ds(off, 128)], iv)
        pltpu.async_copy(uv_hbm.at[iv], rows, sem).wait()
        pltpu.sync_copy(rows, out_hbm.at[pl.ds(off, 128)])
        return carry
    lax.fori_loop(0, 8, blk, 0)


def _prescale_tc(deg_ref, feats_ref, o_ref):
    d = deg_ref[0][:, :1]
    ci = lax.rsqrt(jnp.maximum(d, 1.0))
    o_ref[0] = feats_ref[0] * ci


def _combine_tc(s_ref, deg_ref, w_ref, wfc_ref, bfc_ref, o_ref):
    acc = jnp.zeros((2048, D), jnp.float32)
    for r in range(RR):
        d = deg_ref[r][:, :1]
        cj = lax.rsqrt(jnp.maximum(d, 1.0))
        acc = acc + jnp.dot(s_ref[r] * cj, w_ref[r],
                            preferred_element_type=jnp.float32)
    h = jnp.where(acc > 0, acc, 0.1 * acc)
    z = jnp.dot(h, wfc_ref[...], preferred_element_type=jnp.float32) + bfc_ref[0]
    z = jnp.where(z > 0, z, 0.1 * z)
    o_ref[0] = jnp.concatenate([z, jnp.zeros((2048, D - DO), jnp.float32)], axis=1)


def _decoder_tc(u_ref, v_ref, q_ref, a_ref, o_ref):
    u = u_ref[:, :DO]
    v = v_ref[:, :DO]
    b0 = jnp.sum(jnp.dot(u, q_ref[0], preferred_element_type=jnp.float32) * v,
                 axis=1)
    b1 = jnp.sum(jnp.dot(u, q_ref[1], preferred_element_type=jnp.float32) * v,
                 axis=1)
    o_ref[...] = b0[:, None] * a_ref[0][None, :] + b1[:, None] * a_ref[1][None, :]


def kernel(head_enc, tail_enc, ufeat, ifeat, head_id, tail_id, W, W_fc, b_fc,
           Q, a_comb):
    f32 = jnp.float32
    src_all = jnp.concatenate([head_enc[0], tail_enc[0]]).astype(jnp.int32)
    dst_all = jnp.concatenate([head_enc[1], tail_enc[1]]).astype(jnp.int32)
    qoff = jnp.repeat(jnp.arange(10, dtype=jnp.int32) * N, EPC)
    srcb = src_all + qoff
    dstb = dst_all
    hist_idx = jnp.concatenate([src_all, dst_all])
    feats = jnp.stack([ifeat, ufeat])
    ones16 = jnp.ones((128, 16), f32)
    zeros16 = jnp.zeros((ROWS_A, 16), f32)
    zeros128 = jnp.zeros((ROWS_A, D), f32)
    ones80 = jnp.ones((80, D), f32)
    pair_idx = jnp.concatenate([head_id.astype(jnp.int32),
                                tail_id.astype(jnp.int32) + N])

    mesh = _sc_mesh()

    deg2 = pl.kernel(
        _degrees_sc,
        out_type=jax.ShapeDtypeStruct((20, N, D), f32),
        mesh=mesh,
        scratch_types=[
            pltpu.VMEM_SHARED((N, D), f32),
            pltpu.VMEM((2, 80), jnp.int32),
            pltpu.VMEM((80, D), f32),
            pltpu.SemaphoreType.DMA((2,)),
        ],
    )(hist_idx, ones80, zeros128)


    fscaled = pl.pallas_call(
        _prescale_tc,
        grid=(10, 5),
        in_specs=[
            pl.BlockSpec((1, 2048, D), lambda q, b: (q, b, 0)),
            pl.BlockSpec((1, 2048, D), lambda q, b: (q // 5, b, 0)),
        ],
        out_specs=pl.BlockSpec((1, 2048, D), lambda q, b: (q, b, 0)),
        out_shape=jax.ShapeDtypeStruct((10, N, D), f32),
    )(deg2, feats)

    S = pl.kernel(
        _segsum_sc,
        out_type=jax.ShapeDtypeStruct((10, N, D), f32),
        mesh=mesh,
        scratch_types=[
            pltpu.VMEM_SHARED((N, D), f32),
            pltpu.VMEM((2, 80), jnp.int32),
            pltpu.VMEM((2, 80), jnp.int32),
            pltpu.VMEM((2, 80, D), f32),
            pltpu.SemaphoreType.DMA((2,)),
            pltpu.SemaphoreType.DMA((2,)),
        ],
    )(fscaled.reshape(10 * N, D), srcb, dstb, zeros128)

    outs = pl.pallas_call(
        _combine_tc,
        grid=(2, 5),
        in_specs=[
            pl.BlockSpec((RR, 2048, D), lambda c, b: (c, b, 0)),
            pl.BlockSpec((RR, 2048, D), lambda c, b: (2 + c, b, 0)),
            pl.BlockSpec((RR, D, D), lambda c, b: (0, 0, 0)),
            pl.BlockSpec((D, DO), lambda c, b: (0, 0)),
            pl.BlockSpec((1, DO), lambda c, b: (0, 0)),
        ],
        out_specs=pl.BlockSpec((1, 2048, D), lambda c, b: (c, b, 0)),
        out_shape=jax.ShapeDtypeStruct((2, N, D), f32),
    )(S, deg2, W, W_fc, b_fc.reshape(1, DO))

    uv = pl.kernel(
        _pairgather_sc,
        out_type=jax.ShapeDtypeStruct((2 * B, D), f32),
        mesh=mesh,
        scratch_types=[
            pltpu.VMEM((128,), jnp.int32),
            pltpu.VMEM((128, D), f32),
            pltpu.SemaphoreType.DMA,
        ],
    )(outs.reshape(2 * N, D), pair_idx)

    pred = pl.pallas_call(
        _decoder_tc,
        grid=(8,),
        in_specs=[
            pl.BlockSpec((2048, D), lambda i: (i, 0)),
            pl.BlockSpec((2048, D), lambda i: (i + 8, 0)),
            pl.BlockSpec((2, DO, DO), lambda i: (0, 0, 0)),
            pl.BlockSpec((2, RR), lambda i: (0, 0)),
        ],
        out_specs=pl.BlockSpec((2048, RR), lambda i: (i, 0)),
        out_shape=jax.ShapeDtypeStruct((B, RR), f32),
    )(uv, uv, Q, a_comb)

    return pred


# speedup vs baseline: 9.0933x; 1.1427x over previous
import jax
import jax.numpy as jnp
from jax import lax
from jax.experimental import pallas as pl
from jax.experimental.pallas import tpu as pltpu
from jax.experimental.pallas import tpu_sc as plsc

N = 10000
E = 320000
RR = 5
EPC = E // RR
D = 128
DO = 64
B = 16384
NBLK = EPC // 128
NTILES = 16
ROWS_A = 640
ROWS_B = 400


def _sc_mesh():
    return plsc.VectorSubcoreMesh(core_axis_name="c", subcore_axis_name="s")


def _split_rows(sid, do_copy):
    @pl.when(sid < 15)
    def _main():
        do_copy(sid * ROWS_A, ROWS_A)

    @pl.when(sid == 15)
    def _tail():
        do_copy(15 * ROWS_A, ROWS_B)


def _degrees_sc(hist_hbm, ones_hbm, zeros_hbm, deg_hbm, shared, iv, ones_v, sems):
    cid = lax.axis_index("c")
    sid = lax.axis_index("s")
    pltpu.sync_copy(ones_hbm, ones_v)

    def zero_my(st, cnt):
        pltpu.sync_copy(zeros_hbm.at[pl.ds(0, cnt)],
                        shared.at[pl.ds(st, cnt)])
    _split_rows(sid, zero_my)
    plsc.subcore_barrier()

    NBK = 50
    BW = 80

    def wait_scat(buf):
        pltpu.make_async_copy(ones_v, shared.at[iv.at[buf]],
                              sems.at[buf]).wait()

    def hist_body(hl, carry):
        h = cid * 10 + hl
        base = pl.multiple_of(h * EPC + sid * (EPC // 16), 8)
        pltpu.sync_copy(hist_hbm.at[pl.ds(base, BW)], iv.at[0])

        def blk(k, c2):
            buf = lax.rem(k, 2)
            nbuf = lax.rem(k + 1, 2)
            pltpu.async_copy(ones_v, shared.at[iv.at[buf]], sems.at[buf],
                             add=True)

            @pl.when(k < NBK - 1)
            def _prefetch():
                @pl.when(k >= 1)
                def _drain():
                    wait_scat(nbuf)
                off = pl.multiple_of(base + (k + 1) * BW, 8)
                pltpu.sync_copy(hist_hbm.at[pl.ds(off, BW)], iv.at[nbuf])
            return c2
        lax.fori_loop(0, NBK, blk, 0)
        wait_scat(0)
        wait_scat(1)

        plsc.subcore_barrier()

        def out_my(st, cnt):
            pltpu.sync_copy(shared.at[pl.ds(st, cnt)],
                            deg_hbm.at[h, pl.ds(st, cnt)])
            pltpu.sync_copy(zeros_hbm.at[pl.ds(0, cnt)],
                            shared.at[pl.ds(st, cnt)])
        _split_rows(sid, out_my)
        plsc.subcore_barrier()
        return carry
    lax.fori_loop(0, 10, hist_body, 0)


def _segsum_sc(ftab_hbm, srcb_hbm, dstb_hbm, zeros_hbm, s_hbm,
               shared, sivb, divb, rows, gsem, ssem):
    cid = lax.axis_index("c")
    sid = lax.axis_index("s")

    def zero_my(start, cnt):
        pltpu.sync_copy(zeros_hbm.at[pl.ds(0, cnt)],
                        shared.at[pl.ds(start, cnt)])
    _split_rows(sid, zero_my)
    plsc.subcore_barrier()

    NBK = 50

    def chunk_body(hl, carry):
        q = cid * 5 + hl
        pltpu.sync_copy(srcb_hbm.at[q, sid], sivb)
        pltpu.sync_copy(dstb_hbm.at[q, sid], divb)

        def gather_start(k, buf):
            pltpu.async_copy(ftab_hbm.at[sivb.at[k]], rows.at[buf],
                             gsem.at[buf])

        def wait_scat(buf):
            pltpu.make_async_copy(rows.at[buf], shared.at[divb.at[0]],
                                  ssem.at[buf]).wait()

        gather_start(0, 0)

        def blk(k, c2):
            buf = lax.rem(k, 2)
            nbuf = lax.rem(k + 1, 2)

            @pl.when(k < NBK - 1)
            def _prefetch():
                @pl.when(k >= 1)
                def _drain():
                    wait_scat(nbuf)
                gather_start(k + 1, nbuf)

            pltpu.make_async_copy(ftab_hbm.at[sivb.at[k]], rows.at[buf],
                                  gsem.at[buf]).wait()
            pltpu.async_copy(rows.at[buf], shared.at[divb.at[k]],
                             ssem.at[buf], add=True)
            return c2
        lax.fori_loop(0, NBK, blk, 0)
        wait_scat(0)
        wait_scat(1)

        plsc.subcore_barrier()

        def out_my(start, cnt):
            pltpu.sync_copy(shared.at[pl.ds(start, cnt)],
                            s_hbm.at[q, pl.ds(start, cnt)])
            pltpu.sync_copy(zeros_hbm.at[pl.ds(0, cnt)],
                            shared.at[pl.ds(start, cnt)])
        _split_rows(sid, out_my)
        plsc.subcore_barrier()
        return carry
    lax.fori_loop(0, 5, chunk_body, 0)


def _pairgather_sc(uv_hbm, idx_hbm, out_hbm, iv, rows, sem):
    cid = lax.axis_index("c")
    sid = lax.axis_index("s")
    wid = cid * NTILES + sid

    def blk(k, carry):
        j = wid * 8 + k
        off = pl.multiple_of(j * 128, 128)
        pltpu.sync_copy(idx_hbm.at[pl.ds(off, 128)], iv)
        pltpu.async_copy(uv_hbm.at[iv], rows, sem).wait()
        pltpu.sync_copy(rows, out_hbm.at[pl.ds(off, 128)])
        return carry
    lax.fori_loop(0, 8, blk, 0)


def _prescale_tc(deg_ref, feats_ref, o_ref):
    d = deg_ref[0][:, :1]
    ci = lax.rsqrt(jnp.maximum(d, 1.0))
    o_ref[0] = feats_ref[0] * ci


def _combine_tc(s_ref, deg_ref, w_ref, wfc_ref, bfc_ref, o_ref):
    acc = jnp.zeros((2048, D), jnp.float32)
    for r in range(RR):
        d = deg_ref[r][:, :1]
        cj = lax.rsqrt(jnp.maximum(d, 1.0))
        acc = acc + jnp.dot(s_ref[r] * cj, w_ref[r],
                            preferred_element_type=jnp.float32)
    h = jnp.where(acc > 0, acc, 0.1 * acc)
    z = jnp.dot(h, wfc_ref[...], preferred_element_type=jnp.float32) + bfc_ref[0]
    z = jnp.where(z > 0, z, 0.1 * z)
    o_ref[0] = jnp.concatenate([z, jnp.zeros((2048, D - DO), jnp.float32)], axis=1)


def _decoder_tc(u_ref, v_ref, q_ref, a_ref, o_ref):
    u = u_ref[:, :DO]
    v = v_ref[:, :DO]
    b0 = jnp.sum(jnp.dot(u, q_ref[0], preferred_element_type=jnp.float32) * v,
                 axis=1)
    b1 = jnp.sum(jnp.dot(u, q_ref[1], preferred_element_type=jnp.float32) * v,
                 axis=1)
    o_ref[...] = b0[:, None] * a_ref[0][None, :] + b1[:, None] * a_ref[1][None, :]


def kernel(head_enc, tail_enc, ufeat, ifeat, head_id, tail_id, W, W_fc, b_fc,
           Q, a_comb):
    f32 = jnp.float32
    src_all = jnp.concatenate([head_enc[0], tail_enc[0]]).astype(jnp.int32)
    dst_all = jnp.concatenate([head_enc[1], tail_enc[1]]).astype(jnp.int32)
    qoff = jnp.repeat(jnp.arange(10, dtype=jnp.int32) * N, EPC)
    srcb = src_all + qoff
    dstb = dst_all
    hist_idx = jnp.concatenate([src_all, dst_all])
    feats = jnp.stack([ifeat, ufeat])
    ones16 = jnp.ones((128, 16), f32)
    zeros16 = jnp.zeros((ROWS_A, 16), f32)
    zeros128 = jnp.zeros((ROWS_A, D), f32)
    ones80 = jnp.ones((80, D), f32)
    pair_idx = jnp.concatenate([head_id.astype(jnp.int32),
                                tail_id.astype(jnp.int32) + N])

    mesh = _sc_mesh()

    deg2 = pl.kernel(
        _degrees_sc,
        out_type=jax.ShapeDtypeStruct((20, N, D), f32),
        mesh=mesh,
        scratch_types=[
            pltpu.VMEM_SHARED((N, D), f32),
            pltpu.VMEM((2, 80), jnp.int32),
            pltpu.VMEM((80, D), f32),
            pltpu.SemaphoreType.DMA((2,)),
        ],
    )(hist_idx, ones80, zeros128)


    fscaled = pl.pallas_call(
        _prescale_tc,
        grid=(10, 5),
        in_specs=[
            pl.BlockSpec((1, 2048, D), lambda q, b: (q, b, 0)),
            pl.BlockSpec((1, 2048, D), lambda q, b: (q // 5, b, 0)),
        ],
        out_specs=pl.BlockSpec((1, 2048, D), lambda q, b: (q, b, 0)),
        out_shape=jax.ShapeDtypeStruct((10, N, D), f32),
    )(deg2, feats)

    S = pl.kernel(
        _segsum_sc,
        out_type=jax.ShapeDtypeStruct((10, N, D), f32),
        mesh=mesh,
        scratch_types=[
            pltpu.VMEM_SHARED((N, D), f32),
            pltpu.VMEM((50, 80), jnp.int32),
            pltpu.VMEM((50, 80), jnp.int32),
            pltpu.VMEM((2, 80, D), f32),
            pltpu.SemaphoreType.DMA((2,)),
            pltpu.SemaphoreType.DMA((2,)),
        ],
    )(fscaled.reshape(10 * N, D), srcb.reshape(10, 16, 50, 80),
      dstb.reshape(10, 16, 50, 80), zeros128)

    outs = pl.pallas_call(
        _combine_tc,
        grid=(2, 5),
        in_specs=[
            pl.BlockSpec((RR, 2048, D), lambda c, b: (c, b, 0)),
            pl.BlockSpec((RR, 2048, D), lambda c, b: (2 + c, b, 0)),
            pl.BlockSpec((RR, D, D), lambda c, b: (0, 0, 0)),
            pl.BlockSpec((D, DO), lambda c, b: (0, 0)),
            pl.BlockSpec((1, DO), lambda c, b: (0, 0)),
        ],
        out_specs=pl.BlockSpec((1, 2048, D), lambda c, b: (c, b, 0)),
        out_shape=jax.ShapeDtypeStruct((2, N, D), f32),
    )(S, deg2, W, W_fc, b_fc.reshape(1, DO))

    uv = pl.kernel(
        _pairgather_sc,
        out_type=jax.ShapeDtypeStruct((2 * B, D), f32),
        mesh=mesh,
        scratch_types=[
            pltpu.VMEM((128,), jnp.int32),
            pltpu.VMEM((128, D), f32),
            pltpu.SemaphoreType.DMA,
        ],
    )(outs.reshape(2 * N, D), pair_idx)

    pred = pl.pallas_call(
        _decoder_tc,
        grid=(8,),
        in_specs=[
            pl.BlockSpec((2048, D), lambda i: (i, 0)),
            pl.BlockSpec((2048, D), lambda i: (i + 8, 0)),
            pl.BlockSpec((2, DO, DO), lambda i: (0, 0, 0)),
            pl.BlockSpec((2, RR), lambda i: (0, 0)),
        ],
        out_specs=pl.BlockSpec((2048, RR), lambda i: (i, 0)),
        out_shape=jax.ShapeDtypeStruct((B, RR), f32),
    )(uv, uv, Q, a_comb)

    return pred


# speedup vs baseline: 9.1025x; 1.0010x over previous
import jax
import jax.numpy as jnp
from jax import lax
from jax.experimental import pallas as pl
from jax.experimental.pallas import tpu as pltpu
from jax.experimental.pallas import tpu_sc as plsc

N = 10000
E = 320000
RR = 5
EPC = E // RR
D = 128
DO = 64
B = 16384
NTILES = 16
ROWS_A = 640
ROWS_B = 400


def _sc_mesh():
    return plsc.VectorSubcoreMesh(core_axis_name="c", subcore_axis_name="s")


def _split_rows(sid, do_copy):
    @pl.when(sid < 15)
    def _main():
        do_copy(sid * ROWS_A, ROWS_A)

    @pl.when(sid == 15)
    def _tail():
        do_copy(15 * ROWS_A, ROWS_B)


def _degrees_sc(hist_hbm, ones_hbm, zeros_hbm, deg_hbm, shared, iv, ones_v, sems):
    cid = lax.axis_index("c")
    sid = lax.axis_index("s")
    pltpu.sync_copy(ones_hbm, ones_v)

    def zero_my(st, cnt):
        pltpu.sync_copy(zeros_hbm.at[pl.ds(0, cnt)],
                        shared.at[pl.ds(st, cnt)])
    _split_rows(sid, zero_my)
    plsc.subcore_barrier()

    NBK = 50
    BW = 80

    def wait_scat(buf):
        pltpu.make_async_copy(ones_v, shared.at[iv.at[buf]],
                              sems.at[buf]).wait()

    def hist_body(hl, carry):
        h = cid * 10 + hl
        base = pl.multiple_of(h * EPC + sid * (EPC // 16), 8)
        pltpu.sync_copy(hist_hbm.at[pl.ds(base, BW)], iv.at[0])

        def blk(k, c2):
            buf = lax.rem(k, 2)
            nbuf = lax.rem(k + 1, 2)
            pltpu.async_copy(ones_v, shared.at[iv.at[buf]], sems.at[buf],
                             add=True)

            @pl.when(k < NBK - 1)
            def _prefetch():
                @pl.when(k >= 1)
                def _drain():
                    wait_scat(nbuf)
                off = pl.multiple_of(base + (k + 1) * BW, 8)
                pltpu.sync_copy(hist_hbm.at[pl.ds(off, BW)], iv.at[nbuf])
            return c2
        lax.fori_loop(0, NBK, blk, 0)
        wait_scat(0)
        wait_scat(1)

        plsc.subcore_barrier()

        def out_my(st, cnt):
            pltpu.sync_copy(shared.at[pl.ds(st, cnt)],
                            deg_hbm.at[h, pl.ds(st, cnt)])
            pltpu.sync_copy(zeros_hbm.at[pl.ds(0, cnt)],
                            shared.at[pl.ds(st, cnt)])
        _split_rows(sid, out_my)
        plsc.subcore_barrier()
        return carry
    lax.fori_loop(0, 10, hist_body, 0)


def _segsum_sc(ftab_hbm, srcb_hbm, dstb_hbm, zeros_hbm, s_hbm,
               shared, sivb, divb, rows, gsem, ssem):
    cid = lax.axis_index("c")
    sid = lax.axis_index("s")

    def zero_my(start, cnt):
        pltpu.sync_copy(zeros_hbm.at[pl.ds(0, cnt)],
                        shared.at[pl.ds(start, cnt)])
    _split_rows(sid, zero_my)
    plsc.subcore_barrier()

    NBK = 50

    def chunk_body(hl, carry):
        q = cid * 5 + hl
        pltpu.sync_copy(srcb_hbm.at[q, sid], sivb)
        pltpu.sync_copy(dstb_hbm.at[q, sid], divb)

        def gather_start(k, buf):
            pltpu.async_copy(ftab_hbm.at[sivb.at[k]], rows.at[buf],
                             gsem.at[buf])

        def wait_scat(buf):
            pltpu.make_async_copy(rows.at[buf], shared.at[divb.at[0]],
                                  ssem.at[buf]).wait()

        gather_start(0, 0)

        def blk(k, c2):
            buf = lax.rem(k, 2)
            nbuf = lax.rem(k + 1, 2)

            @pl.when(k < NBK - 1)
            def _prefetch():
                @pl.when(k >= 1)
                def _drain():
                    wait_scat(nbuf)
                gather_start(k + 1, nbuf)

            pltpu.make_async_copy(ftab_hbm.at[sivb.at[k]], rows.at[buf],
                                  gsem.at[buf]).wait()
            pltpu.async_copy(rows.at[buf], shared.at[divb.at[k]],
                             ssem.at[buf], add=True)
            return c2
        lax.fori_loop(0, NBK, blk, 0)
        wait_scat(0)
        wait_scat(1)

        plsc.subcore_barrier()

        def out_my(start, cnt):
            pltpu.sync_copy(shared.at[pl.ds(start, cnt)],
                            s_hbm.at[q, pl.ds(start, cnt)])
            pltpu.sync_copy(zeros_hbm.at[pl.ds(0, cnt)],
                            shared.at[pl.ds(start, cnt)])
        _split_rows(sid, out_my)
        plsc.subcore_barrier()
        return carry
    lax.fori_loop(0, 5, chunk_body, 0)


def _pairgather_sc(uv_hbm, idx_hbm, out_hbm, iv, rows, sem):
    cid = lax.axis_index("c")
    sid = lax.axis_index("s")
    wid = cid * NTILES + sid

    def blk(k, carry):
        j = wid * 8 + k
        off = pl.multiple_of(j * 128, 128)
        pltpu.sync_copy(idx_hbm.at[pl.ds(off, 128)], iv)
        pltpu.async_copy(uv_hbm.at[iv], rows, sem).wait()
        pltpu.sync_copy(rows, out_hbm.at[pl.ds(off, 128)])
        return carry
    lax.fori_loop(0, 8, blk, 0)


def _prescale_tc(deg_ref, feats_ref, o_ref):
    d = deg_ref[0][:, :1]
    ci = lax.rsqrt(jnp.maximum(d, 1.0))
    o_ref[0] = feats_ref[0] * ci


def _combine_tc(s_ref, deg_ref, w_ref, wfc_ref, bfc_ref, o_ref):
    acc = jnp.zeros((2048, D), jnp.float32)
    for r in range(RR):
        d = deg_ref[r][:, :1]
        cj = lax.rsqrt(jnp.maximum(d, 1.0))
        acc = acc + jnp.dot(s_ref[r] * cj, w_ref[r],
                            preferred_element_type=jnp.float32)
    h = jnp.where(acc > 0, acc, 0.1 * acc)
    z = jnp.dot(h, wfc_ref[...], preferred_element_type=jnp.float32) + bfc_ref[0]
    z = jnp.where(z > 0, z, 0.1 * z)
    o_ref[0] = jnp.concatenate([z, jnp.zeros((2048, D - DO), jnp.float32)], axis=1)


def _decoder_tc(u_ref, v_ref, q_ref, a_ref, o_ref):
    u = u_ref[:, :DO]
    v = v_ref[:, :DO]
    b0 = jnp.sum(jnp.dot(u, q_ref[0], preferred_element_type=jnp.float32) * v,
                 axis=1)
    b1 = jnp.sum(jnp.dot(u, q_ref[1], preferred_element_type=jnp.float32) * v,
                 axis=1)
    o_ref[...] = b0[:, None] * a_ref[0][None, :] + b1[:, None] * a_ref[1][None, :]


def kernel(head_enc, tail_enc, ufeat, ifeat, head_id, tail_id, W, W_fc, b_fc,
           Q, a_comb):
    f32 = jnp.float32
    src_all = jnp.concatenate([head_enc[0], tail_enc[0]]).astype(jnp.int32)
    dst_all = jnp.concatenate([head_enc[1], tail_enc[1]]).astype(jnp.int32)
    qoff = jnp.repeat(jnp.arange(10, dtype=jnp.int32) * N, EPC)
    srcb = src_all + qoff
    dstb = dst_all
    hist_idx = jnp.concatenate([src_all, dst_all])
    feats = jnp.stack([ifeat, ufeat])
    zeros128 = jnp.zeros((ROWS_A, D), f32)
    ones80 = jnp.ones((80, D), f32)
    pair_idx = jnp.concatenate([head_id.astype(jnp.int32),
                                tail_id.astype(jnp.int32) + N])

    mesh = _sc_mesh()

    deg2 = pl.kernel(
        _degrees_sc,
        out_type=jax.ShapeDtypeStruct((20, N, D), f32),
        mesh=mesh,
        scratch_types=[
            pltpu.VMEM_SHARED((N, D), f32),
            pltpu.VMEM((2, 80), jnp.int32),
            pltpu.VMEM((80, D), f32),
            pltpu.SemaphoreType.DMA((2,)),
        ],
    )(hist_idx, ones80, zeros128)


    fscaled = pl.pallas_call(
        _prescale_tc,
        grid=(10, 5),
        in_specs=[
            pl.BlockSpec((1, 2048, D), lambda q, b: (q, b, 0)),
            pl.BlockSpec((1, 2048, D), lambda q, b: (q // 5, b, 0)),
        ],
        out_specs=pl.BlockSpec((1, 2048, D), lambda q, b: (q, b, 0)),
        out_shape=jax.ShapeDtypeStruct((10, N, D), f32),
    )(deg2, feats)

    S = pl.kernel(
        _segsum_sc,
        out_type=jax.ShapeDtypeStruct((10, N, D), f32),
        mesh=mesh,
        scratch_types=[
            pltpu.VMEM_SHARED((N, D), f32),
            pltpu.VMEM((50, 80), jnp.int32),
            pltpu.VMEM((50, 80), jnp.int32),
            pltpu.VMEM((2, 80, D), f32),
            pltpu.SemaphoreType.DMA((2,)),
            pltpu.SemaphoreType.DMA((2,)),
        ],
    )(fscaled.reshape(10 * N, D), srcb.reshape(10, 16, 50, 80),
      dstb.reshape(10, 16, 50, 80), zeros128)

    outs = pl.pallas_call(
        _combine_tc,
        grid=(2, 5),
        in_specs=[
            pl.BlockSpec((RR, 2048, D), lambda c, b: (c, b, 0)),
            pl.BlockSpec((RR, 2048, D), lambda c, b: (2 + c, b, 0)),
            pl.BlockSpec((RR, D, D), lambda c, b: (0, 0, 0)),
            pl.BlockSpec((D, DO), lambda c, b: (0, 0)),
            pl.BlockSpec((1, DO), lambda c, b: (0, 0)),
        ],
        out_specs=pl.BlockSpec((1, 2048, D), lambda c, b: (c, b, 0)),
        out_shape=jax.ShapeDtypeStruct((2, N, D), f32),
    )(S, deg2, W, W_fc, b_fc.reshape(1, DO))

    uv = pl.kernel(
        _pairgather_sc,
        out_type=jax.ShapeDtypeStruct((2 * B, D), f32),
        mesh=mesh,
        scratch_types=[
            pltpu.VMEM((128,), jnp.int32),
            pltpu.VMEM((128, D), f32),
            pltpu.SemaphoreType.DMA,
        ],
    )(outs.reshape(2 * N, D), pair_idx)

    pred = pl.pallas_call(
        _decoder_tc,
        grid=(8,),
        in_specs=[
            pl.BlockSpec((2048, D), lambda i: (i, 0)),
            pl.BlockSpec((2048, D), lambda i: (i + 8, 0)),
            pl.BlockSpec((2, DO, DO), lambda i: (0, 0, 0)),
            pl.BlockSpec((2, RR), lambda i: (0, 0)),
        ],
        out_specs=pl.BlockSpec((2048, RR), lambda i: (i, 0)),
        out_shape=jax.ShapeDtypeStruct((B, RR), f32),
    )(uv, uv, Q, a_comb)

    return pred
